# baseline (speedup 1.0000x reference)
import functools

import jax
import jax.numpy as jnp
from jax.experimental import pallas as pl
from jax.experimental.pallas import tpu as pltpu

_B, _T, _D_IN, _G, _D = 8, 2048, 512, 8, 256
_N = _B * _T
_BLK = 512
_K_AUG = _G * _D_IN + _G


def _fused_body(x_ref, wpi_ref, bpi_ref, wsig_ref, wmu_ref, mu_ref, sig_ref):
    x = x_ref[...]
    logits = jnp.dot(x, wpi_ref[...], preferred_element_type=jnp.float32)
    logits = logits + bpi_ref[...]
    g = jnp.argmax(logits, axis=1).astype(jnp.int32)[:, None]

    xh = x.astype(jnp.bfloat16)
    zero = jnp.zeros_like(xh)
    parts = [jnp.where(g == k, xh, zero) for k in range(_G)]
    onehot = (g == jax.lax.broadcasted_iota(jnp.int32, (1, _G), 1)
              ).astype(jnp.bfloat16)
    x_aug = jnp.concatenate(parts + [jnp.broadcast_to(onehot, (_BLK, _G))],
                            axis=1)

    mu_ref[...] = jnp.dot(x_aug, wmu_ref[...],
                          preferred_element_type=jnp.float32)
    sig_ref[...] = jnp.exp(jnp.dot(x_aug, wsig_ref[...],
                                   preferred_element_type=jnp.float32))


def _augment_weights(W, b):
    blocks = W.reshape(_D_IN, _G, _D).transpose(1, 0, 2).reshape(_G * _D_IN, _D)
    return jnp.concatenate([blocks, b.reshape(_G, _D)], axis=0
                           ).astype(jnp.bfloat16)


@jax.jit
def kernel(x, W_pi, b_pi, W_sigma, b_sigma, W_mu, b_mu):
    xf = x.reshape(_N, _D_IN)
    wsig_aug = _augment_weights(W_sigma, b_sigma)
    wmu_aug = _augment_weights(W_mu, b_mu)
    grid = (_N // _BLK,)
    full = lambda i: (0, 0)
    mu, sig = pl.pallas_call(
        _fused_body,
        grid=grid,
        in_specs=[
            pl.BlockSpec((_BLK, _D_IN), lambda i: (i, 0)),
            pl.BlockSpec((_D_IN, _G), full),
            pl.BlockSpec((_G,), lambda i: (0,)),
            pl.BlockSpec((_K_AUG, _D), full),
            pl.BlockSpec((_K_AUG, _D), full),
        ],
        out_specs=[
            pl.BlockSpec((_BLK, _D), lambda i: (i, 0)),
            pl.BlockSpec((_BLK, _D), lambda i: (i, 0)),
        ],
        out_shape=[
            jax.ShapeDtypeStruct((_N, _D), jnp.float32),
            jax.ShapeDtypeStruct((_N, _D), jnp.float32),
        ],
        compiler_params=pltpu.CompilerParams(
            dimension_semantics=("arbitrary",),
        ),
    )(xf, W_pi, b_pi, wsig_aug, wmu_aug)
    return mu.reshape(_B, _T, _D), sig.reshape(_B, _T, _D)
